# baseline (speedup 1.0000x reference)
import functools

import jax
import jax.numpy as jnp
from jax import lax
from jax.experimental import pallas as pl
from jax.experimental.pallas import tpu as pltpu
from jax.experimental.pallas import tpu_sc as plsc

N = 10000
E = 320000
D = 128
H1 = 64
H2 = 32

NC = 2
NS = 16
NW = NC * NS

C = 128
CH_PW = 80
HALF = 40
K0 = 128
K1 = 160 - K0
KMAXH = 64
NCH = NW * CH_PW
EP = NCH * C
NP = 10240
DEG_T = NP // NS

NA = N + 8
RT_N = 632
RL_N = N - (NS - 1) * RT_N
RT_Z = 632
RL_Z = NA - (NS - 1) * RT_Z

_MESH = plsc.VectorSubcoreMesh(core_axis_name="c", subcore_axis_name="s")
f32 = jnp.float32


def _fanout(s, rt, rl, fn):
    @pl.when(s < NS - 1)
    def _():
        fn(pl.multiple_of(s * rt, 8), rt)

    @pl.when(s == NS - 1)
    def _():
        fn(pl.multiple_of((NS - 1) * rt, 8), rl)


def _deg_body(e2d, ones_h, zz, out, dst_v, ones_v, acc):
    c = lax.axis_index("c")
    s = lax.axis_index("s")
    wid = s * NC + c
    pltpu.sync_copy(ones_h, ones_v)
    doff = pl.multiple_of(s * DEG_T, 8)
    pltpu.sync_copy(zz.at[pl.ds(doff, DEG_T)], acc.at[pl.ds(doff, DEG_T)])
    coff = pl.multiple_of(wid * CH_PW, 8)
    pltpu.sync_copy(e2d.at[1, pl.ds(coff, CH_PW)], dst_v)
    plsc.subcore_barrier()

    def step(j, carry):
        pltpu.sync_copy(ones_v, acc.at[dst_v.at[j]], add=True)
        return carry

    lax.fori_loop(0, CH_PW, step, 0)
    plsc.subcore_barrier()
    pltpu.sync_copy(acc.at[pl.ds(doff, DEG_T)],
                    out.at[c, pl.ds(doff, DEG_T)])


_deg_call = pl.kernel(
    _deg_body,
    out_type=jax.ShapeDtypeStruct((NC, NP), f32),
    mesh=_MESH,
    scratch_types=[
        pltpu.VMEM((CH_PW, C), jnp.int32),
        pltpu.VMEM((C,), f32),
        pltpu.VMEM_SHARED((NP,), f32),
    ],
)


FP = 128


def _agg_body(xs, e2d, zz, out, src_v, dst_v, rows2, acc,
              gsem0, gsem1, ssem0, ssem1):
    c = lax.axis_index("c")
    s = lax.axis_index("s")
    wid = s * NC + c
    _fanout(s, RT_Z, RL_Z, lambda o, n: pltpu.sync_copy(
        zz.at[pl.ds(o, n)], acc.at[pl.ds(o, n)]))
    plsc.subcore_barrier()
    b0 = rows2.at[0]
    b1 = rows2.at[1]

    def gat(j, buf, sem):
        return pltpu.async_copy(xs.at[src_v.at[j]], buf, sem)

    def gat_wait(j, buf, sem):
        pltpu.make_async_copy(xs.at[src_v.at[j]], buf, sem).wait()

    def sca(j, buf, sem):
        return pltpu.async_copy(buf, acc.at[dst_v.at[j]], sem, add=True)

    def sca_wait(j, buf, sem):
        pltpu.make_async_copy(xs.at[src_v.at[j]], buf, sem).wait()

    def run_edges(cbase, half):
        for h in range(2):
            coff = pl.multiple_of(cbase + h * half, 8)
            pltpu.sync_copy(e2d.at[0, pl.ds(coff, half)], src_v.at[pl.ds(0, half)])
            pltpu.sync_copy(e2d.at[1, pl.ds(coff, half)], dst_v.at[pl.ds(0, half)])
            gat(0, b0, gsem0)

            def step(t, carry):
                j0 = 2 * t
                j1 = 2 * t + 1

                @pl.when(t > 0)
                def _():
                    sca_wait(j0 - 1, b1, ssem1)
                gat(j1, b1, gsem1)
                gat_wait(j0, b0, gsem0)
                sca(j0, b0, ssem0)
                gat_wait(j1, b1, gsem1)
                sca(j1, b1, ssem1)
                sca_wait(j0, b0, ssem0)

                @pl.when(t < half // 2 - 1)
                def _():
                    gat(j0 + 2, b0, gsem0)
                return carry

            lax.fori_loop(0, half // 2, step, 0)
            sca_wait(half - 1, b1, ssem1)

    pair_base = s * (K0 + K1)

    @pl.when(c == 0)
    def _():
        run_edges(pair_base, K0 // 2)

    @pl.when(c == 1)
    def _():
        run_edges(pair_base + K0, K1 // 2)
    plsc.subcore_barrier()
    _fanout(s, RT_N, RL_N, lambda o, n: pltpu.sync_copy(
        acc.at[pl.ds(o, n)], out.at[c, pl.ds(o, n)]))


_agg_call = pl.kernel(
    _agg_body,
    out_type=jax.ShapeDtypeStruct((NC, N, FP), f32),
    mesh=_MESH,
    scratch_types=[
        pltpu.VMEM((KMAXH, C), jnp.int32),
        pltpu.VMEM((KMAXH, C), jnp.int32),
        pltpu.VMEM((2, C, FP), f32),
        pltpu.VMEM_SHARED((NA, FP), f32),
        pltpu.SemaphoreType.DMA,
        pltpu.SemaphoreType.DMA,
        pltpu.SemaphoreType.DMA,
        pltpu.SemaphoreType.DMA,
    ],
)


R = 1000
G = N // R


def _dinv(dp_ref):
    dp = dp_ref[...]
    return lax.rsqrt(dp[:, 0] + dp[:, 1] + 1.0)


def _mm1_body(x_ref, w_ref, dp_ref, o_ref):
    dinv = _dinv(dp_ref)
    xw = jnp.dot(x_ref[...], w_ref[...], preferred_element_type=f32)
    o_ref[:, :H1] = xw * dinv[:, None]
    o_ref[:, H1:] = jnp.zeros((R, FP - H1), f32)


_mm1 = pl.pallas_call(
    _mm1_body,
    grid=(G,),
    in_specs=[
        pl.BlockSpec((R, D), lambda i: (i, 0)),
        pl.BlockSpec((D, H1), lambda i: (0, 0)),
        pl.BlockSpec((R, NC), lambda i: (i, 0)),
    ],
    out_specs=pl.BlockSpec((R, FP), lambda i: (i, 0)),
    out_shape=jax.ShapeDtypeStruct((N, FP), f32),
)


def _mm2_body(ag_ref, xs_ref, dp_ref, w_ref, b_ref, o_ref):
    dinv = _dinv(dp_ref)
    agg = ag_ref[0, :, :H1] + ag_ref[1, :, :H1] + xs_ref[:, :H1]
    h = jnp.maximum(agg * dinv[:, None] + b_ref[...], 0.0)
    xw = jnp.dot(h, w_ref[...], preferred_element_type=f32)
    o_ref[:, :H2] = xw * dinv[:, None]
    o_ref[:, H2:] = jnp.zeros((R, FP - H2), f32)


_mm2 = pl.pallas_call(
    _mm2_body,
    grid=(G,),
    in_specs=[
        pl.BlockSpec((NC, R, FP), lambda i: (0, i, 0)),
        pl.BlockSpec((R, FP), lambda i: (i, 0)),
        pl.BlockSpec((R, NC), lambda i: (i, 0)),
        pl.BlockSpec((H1, H2), lambda i: (0, 0)),
        pl.BlockSpec((1, H1), lambda i: (0, 0)),
    ],
    out_specs=pl.BlockSpec((R, FP), lambda i: (i, 0)),
    out_shape=jax.ShapeDtypeStruct((N, FP), f32),
)


def _mm3_body(ag_ref, xs_ref, dp_ref, w_ref, b_ref, bfc_ref, o_ref):
    dinv = _dinv(dp_ref)
    agg = ag_ref[0, :, :H2] + ag_ref[1, :, :H2] + xs_ref[:, :H2]
    h = jnp.maximum(agg * dinv[:, None] + b_ref[...], 0.0)
    o_ref[...] = jnp.dot(h, w_ref[...], preferred_element_type=f32) + bfc_ref[...]


_mm3 = pl.pallas_call(
    _mm3_body,
    grid=(G,),
    in_specs=[
        pl.BlockSpec((NC, R, FP), lambda i: (0, i, 0)),
        pl.BlockSpec((R, FP), lambda i: (i, 0)),
        pl.BlockSpec((R, NC), lambda i: (i, 0)),
        pl.BlockSpec((H2, 1), lambda i: (0, 0)),
        pl.BlockSpec((1, H2), lambda i: (0, 0)),
        pl.BlockSpec((1, 1), lambda i: (0, 0)),
    ],
    out_specs=pl.BlockSpec((R, 1), lambda i: (i, 0)),
    out_shape=jax.ShapeDtypeStruct((N, 1), f32),
)


def kernel(x, edge_index, W1, b1, W2, b2, Wfc, bfc):
    i32 = jnp.int32
    pad = jnp.stack([jnp.zeros((EP - E,), i32), jnp.full((EP - E,), N, i32)])
    e2d = jnp.concatenate([edge_index, pad], axis=1).reshape(2, NCH, C)
    z_np = jnp.zeros((NP,), f32)
    z = jnp.zeros((NA, FP), f32)

    degp = _deg_call(e2d, jnp.ones((C,), f32), z_np)[:, :N].T
    xs1 = _mm1(x, W1, degp)
    ag1 = _agg_call(xs1, e2d, z)
    xs2 = _mm2(ag1, xs1, degp, W2, b1.reshape(1, H1))
    ag2 = _agg_call(xs2, e2d, z)
    return _mm3(ag2, xs2, degp, Wfc, b2.reshape(1, H2), bfc.reshape(1, 1))


# speedup vs baseline: 1.0391x; 1.0391x over previous
import functools

import jax
import jax.numpy as jnp
from jax import lax
from jax.experimental import pallas as pl
from jax.experimental.pallas import tpu as pltpu
from jax.experimental.pallas import tpu_sc as plsc

N = 10000
E = 320000
D = 128
H1 = 64
H2 = 32

NC = 2
NS = 16
NW = NC * NS

C = 128
CH_PW = 80
HALF = 40
K0 = 112
K1 = 160 - K0
KMAXH = 56
NCH = NW * CH_PW
EP = NCH * C
NP = 10240
DEG_T = NP // NS

NA = N + 8
RT_N = 632
RL_N = N - (NS - 1) * RT_N
RT_Z = 632
RL_Z = NA - (NS - 1) * RT_Z

_MESH = plsc.VectorSubcoreMesh(core_axis_name="c", subcore_axis_name="s")
f32 = jnp.float32


def _fanout(s, rt, rl, fn):
    @pl.when(s < NS - 1)
    def _():
        fn(pl.multiple_of(s * rt, 8), rt)

    @pl.when(s == NS - 1)
    def _():
        fn(pl.multiple_of((NS - 1) * rt, 8), rl)


def _deg_body(e2d, ones_h, zz, out, dst_v, ones_v, acc):
    c = lax.axis_index("c")
    s = lax.axis_index("s")
    wid = s * NC + c
    pltpu.sync_copy(ones_h, ones_v)
    doff = pl.multiple_of(s * DEG_T, 8)
    pltpu.sync_copy(zz.at[pl.ds(doff, DEG_T)], acc.at[pl.ds(doff, DEG_T)])
    coff = pl.multiple_of(wid * CH_PW, 8)
    pltpu.sync_copy(e2d.at[1, pl.ds(coff, CH_PW)], dst_v)
    plsc.subcore_barrier()

    def step(j, carry):
        pltpu.sync_copy(ones_v, acc.at[dst_v.at[j]], add=True)
        return carry

    lax.fori_loop(0, CH_PW, step, 0)
    plsc.subcore_barrier()
    pltpu.sync_copy(acc.at[pl.ds(doff, DEG_T)],
                    out.at[c, pl.ds(doff, DEG_T)])


_deg_call = pl.kernel(
    _deg_body,
    out_type=jax.ShapeDtypeStruct((NC, NP), f32),
    mesh=_MESH,
    scratch_types=[
        pltpu.VMEM((CH_PW, C), jnp.int32),
        pltpu.VMEM((C,), f32),
        pltpu.VMEM_SHARED((NP,), f32),
    ],
)


FP = 128


def _agg_body(xs, e2d, zz, out, src_v, dst_v, rows2, acc,
              gsem0, gsem1, ssem0, ssem1):
    c = lax.axis_index("c")
    s = lax.axis_index("s")
    wid = s * NC + c
    _fanout(s, RT_Z, RL_Z, lambda o, n: pltpu.sync_copy(
        zz.at[pl.ds(o, n)], acc.at[pl.ds(o, n)]))
    plsc.subcore_barrier()
    b0 = rows2.at[0]
    b1 = rows2.at[1]

    def gat(j, buf, sem):
        return pltpu.async_copy(xs.at[src_v.at[j]], buf, sem)

    def gat_wait(j, buf, sem):
        pltpu.make_async_copy(xs.at[src_v.at[j]], buf, sem).wait()

    def sca(j, buf, sem):
        return pltpu.async_copy(buf, acc.at[dst_v.at[j]], sem, add=True)

    def sca_wait(j, buf, sem):
        pltpu.make_async_copy(xs.at[src_v.at[j]], buf, sem).wait()

    def run_edges(cbase, half):
        for h in range(2):
            coff = pl.multiple_of(cbase + h * half, 8)
            pltpu.sync_copy(e2d.at[0, pl.ds(coff, half)], src_v.at[pl.ds(0, half)])
            pltpu.sync_copy(e2d.at[1, pl.ds(coff, half)], dst_v.at[pl.ds(0, half)])
            gat(0, b0, gsem0)

            def step(t, carry):
                j0 = 2 * t
                j1 = 2 * t + 1

                @pl.when(t > 0)
                def _():
                    sca_wait(j0 - 1, b1, ssem1)
                gat(j1, b1, gsem1)
                gat_wait(j0, b0, gsem0)
                sca(j0, b0, ssem0)
                gat_wait(j1, b1, gsem1)
                sca(j1, b1, ssem1)
                sca_wait(j0, b0, ssem0)

                @pl.when(t < half // 2 - 1)
                def _():
                    gat(j0 + 2, b0, gsem0)
                return carry

            lax.fori_loop(0, half // 2, step, 0)
            sca_wait(half - 1, b1, ssem1)

    pair_base = s * (K0 + K1)

    @pl.when(c == 0)
    def _():
        run_edges(pair_base, K0 // 2)

    @pl.when(c == 1)
    def _():
        run_edges(pair_base + K0, K1 // 2)
    plsc.subcore_barrier()
    _fanout(s, RT_N, RL_N, lambda o, n: pltpu.sync_copy(
        acc.at[pl.ds(o, n)], out.at[c, pl.ds(o, n)]))


_agg_call = pl.kernel(
    _agg_body,
    out_type=jax.ShapeDtypeStruct((NC, N, FP), f32),
    mesh=_MESH,
    scratch_types=[
        pltpu.VMEM((KMAXH, C), jnp.int32),
        pltpu.VMEM((KMAXH, C), jnp.int32),
        pltpu.VMEM((2, C, FP), f32),
        pltpu.VMEM_SHARED((NA, FP), f32),
        pltpu.SemaphoreType.DMA,
        pltpu.SemaphoreType.DMA,
        pltpu.SemaphoreType.DMA,
        pltpu.SemaphoreType.DMA,
    ],
)


R = 1000
G = N // R


def _dinv(dp_ref):
    dp = dp_ref[...]
    return lax.rsqrt(dp[:, 0] + dp[:, 1] + 1.0)


def _mm1_body(x_ref, w_ref, dp_ref, o_ref):
    dinv = _dinv(dp_ref)
    xw = jnp.dot(x_ref[...], w_ref[...], preferred_element_type=f32)
    o_ref[:, :H1] = xw * dinv[:, None]
    o_ref[:, H1:] = jnp.zeros((R, FP - H1), f32)


_mm1 = pl.pallas_call(
    _mm1_body,
    grid=(G,),
    in_specs=[
        pl.BlockSpec((R, D), lambda i: (i, 0)),
        pl.BlockSpec((D, H1), lambda i: (0, 0)),
        pl.BlockSpec((R, NC), lambda i: (i, 0)),
    ],
    out_specs=pl.BlockSpec((R, FP), lambda i: (i, 0)),
    out_shape=jax.ShapeDtypeStruct((N, FP), f32),
)


def _mm2_body(ag_ref, xs_ref, dp_ref, w_ref, b_ref, o_ref):
    dinv = _dinv(dp_ref)
    agg = ag_ref[0, :, :H1] + ag_ref[1, :, :H1] + xs_ref[:, :H1]
    h = jnp.maximum(agg * dinv[:, None] + b_ref[...], 0.0)
    xw = jnp.dot(h, w_ref[...], preferred_element_type=f32)
    o_ref[:, :H2] = xw * dinv[:, None]
    o_ref[:, H2:] = jnp.zeros((R, FP - H2), f32)


_mm2 = pl.pallas_call(
    _mm2_body,
    grid=(G,),
    in_specs=[
        pl.BlockSpec((NC, R, FP), lambda i: (0, i, 0)),
        pl.BlockSpec((R, FP), lambda i: (i, 0)),
        pl.BlockSpec((R, NC), lambda i: (i, 0)),
        pl.BlockSpec((H1, H2), lambda i: (0, 0)),
        pl.BlockSpec((1, H1), lambda i: (0, 0)),
    ],
    out_specs=pl.BlockSpec((R, FP), lambda i: (i, 0)),
    out_shape=jax.ShapeDtypeStruct((N, FP), f32),
)


def _mm3_body(ag_ref, xs_ref, dp_ref, w_ref, b_ref, bfc_ref, o_ref):
    dinv = _dinv(dp_ref)
    agg = ag_ref[0, :, :H2] + ag_ref[1, :, :H2] + xs_ref[:, :H2]
    h = jnp.maximum(agg * dinv[:, None] + b_ref[...], 0.0)
    o_ref[...] = jnp.dot(h, w_ref[...], preferred_element_type=f32) + bfc_ref[...]


_mm3 = pl.pallas_call(
    _mm3_body,
    grid=(G,),
    in_specs=[
        pl.BlockSpec((NC, R, FP), lambda i: (0, i, 0)),
        pl.BlockSpec((R, FP), lambda i: (i, 0)),
        pl.BlockSpec((R, NC), lambda i: (i, 0)),
        pl.BlockSpec((H2, 1), lambda i: (0, 0)),
        pl.BlockSpec((1, H2), lambda i: (0, 0)),
        pl.BlockSpec((1, 1), lambda i: (0, 0)),
    ],
    out_specs=pl.BlockSpec((R, 1), lambda i: (i, 0)),
    out_shape=jax.ShapeDtypeStruct((N, 1), f32),
)


def kernel(x, edge_index, W1, b1, W2, b2, Wfc, bfc):
    i32 = jnp.int32
    pad = jnp.stack([jnp.zeros((EP - E,), i32), jnp.full((EP - E,), N, i32)])
    e2d = jnp.concatenate([edge_index, pad], axis=1).reshape(2, NCH, C)
    z_np = jnp.zeros((NP,), f32)
    z = jnp.zeros((NA, FP), f32)

    degp = _deg_call(e2d, jnp.ones((C,), f32), z_np)[:, :N].T
    xs1 = _mm1(x, W1, degp)
    ag1 = _agg_call(xs1, e2d, z)
    xs2 = _mm2(ag1, xs1, degp, W2, b1.reshape(1, H1))
    ag2 = _agg_call(xs2, e2d, z)
    return _mm3(ag2, xs2, degp, Wfc, b2.reshape(1, H2), bfc.reshape(1, 1))
